# baseline (speedup 1.0000x reference)
import functools

import jax
import jax.numpy as jnp
from jax import lax
from jax.experimental import pallas as pl
from jax.experimental.pallas import tpu as pltpu
from jax.experimental.pallas import tpu_sc as plsc

B = 4096
L = 200
F = 64
VOCAB = 1000000
EPS = 1e-5

NC = 2
NS = 16
NW = NC * NS
BPW = B // NW
IPW = BPW * L

VB = 32768
HVB = VB // 2
NBLK = (VOCAB + VB - 1) // VB
LOBLK = 16
ZPAD = 8

_mesh = plsc.VectorSubcoreMesh(core_axis_name="c", subcore_axis_name="s")


def _pack_body(lastpid):
    def f(tt_ref, o_ref):
        t = jnp.transpose(tt_ref[...])
        th = jnp.concatenate([t[:HVB, :], t[HVB:, :]], axis=1)
        o_ref[...] = jnp.where(pl.program_id(0) == lastpid, 0.0, th)
    return f


def _pack(tableT, nblk, blk0):
    return pl.pallas_call(
        _pack_body(nblk),
        grid=(nblk + 1,),
        in_specs=[pl.BlockSpec(
            (F, VB), lambda j: (0, jnp.minimum(blk0 + j, NBLK - 1)))],
        out_specs=pl.BlockSpec((HVB, 2 * F), lambda j: (j, 0)),
        out_shape=jax.ShapeDtypeStruct((nblk * HVB + ZPAD, 2 * F),
                                       jnp.float32),
    )(tableT)


def _make_gather(loblk, hiblk, nblk, with_partial):
    zrow = nblk * HVB

    def body(refs):
        if with_partial:
            (x_hbm, table_hbm, pin_hbm, out_hbm, idx_v, buf0, buf1, out_v,
             par0, par1, semi, sem0, sem1) = refs
        else:
            (x_hbm, table_hbm, out_hbm, idx_v, buf0, buf1, out_v,
             par0, par1, semi, sem0, sem1) = refs
        wid = lax.axis_index("s") * NC + lax.axis_index("c")
        pltpu.async_copy(x_hbm.at[pl.ds(wid * IPW, IPW)], idx_v, semi).wait()
        if with_partial:
            pltpu.async_copy(pin_hbm.at[pl.ds(wid * BPW * F, BPW * F)],
                             out_v, semi).wait()

        zero = jnp.zeros((16,), jnp.float32)

        def enqueue(r, buf, par, sem):
            base = r * L

            def one(idx, slot):
                h = lax.shift_right_logical(idx, 15)
                valid = jnp.logical_and(h >= loblk, h < hiblk)
                q = (h - loblk) * HVB + (idx & (HVB - 1))
                qe = jnp.where(valid, q, zrow)
                pltpu.async_copy(table_hbm.at[pl.ds(qe, 1)],
                                 buf.at[pl.ds(slot, 1)], sem)
                par[slot] = (lax.shift_right_logical(idx, 14) & 1) * F

            def ek(k, _):
                vec = idx_v[pl.ds(base + k * 16, 16)]
                for t in range(16):
                    one(vec[t], k * 16 + t)
                return 0
            lax.fori_loop(0, (L // 16), ek, 0)
            vec = idx_v[pl.ds(base + L - 16, 16)]
            for t in range(16 - (L % 16), 16):
                one(vec[t], L - 16 + t)

        def drain(buf, sem):
            pltpu.make_async_copy(table_hbm.at[pl.ds(0, L)], buf, sem).wait()

        def accumulate(buf, par, r):
            def jbody(j, carry):
                a0, a1, a2, a3 = carry
                row = buf.at[j]
                off = par[j]
                return (a0 + row[pl.ds(off, 16)],
                        a1 + row[pl.ds(off + 16, 16)],
                        a2 + row[pl.ds(off + 32, 16)],
                        a3 + row[pl.ds(off + 48, 16)])
            if with_partial:
                init = (out_v[pl.ds(r * F, 16)],
                        out_v[pl.ds(r * F + 16, 16)],
                        out_v[pl.ds(r * F + 32, 16)],
                        out_v[pl.ds(r * F + 48, 16)])
            else:
                init = (zero, zero, zero, zero)
            a0, a1, a2, a3 = lax.fori_loop(0, L, jbody, init)
            out_v[pl.ds(r * F, 16)] = a0
            out_v[pl.ds(r * F + 16, 16)] = a1
            out_v[pl.ds(r * F + 32, 16)] = a2
            out_v[pl.ds(r * F + 48, 16)] = a3

        enqueue(0, buf0, par0, sem0)

        def loop(i, _):
            r0 = 2 * i
            enqueue(r0 + 1, buf1, par1, sem1)
            drain(buf0, sem0)
            accumulate(buf0, par0, r0)

            @pl.when(r0 + 2 < BPW)
            def _():
                enqueue(r0 + 2, buf0, par0, sem0)
            drain(buf1, sem1)
            accumulate(buf1, par1, r0 + 1)
            return 0

        lax.fori_loop(0, BPW // 2, loop, 0)
        pltpu.sync_copy(out_v, out_hbm.at[pl.ds(wid * BPW * F, BPW * F)])

    def fn(*refs):
        body(refs)

    return functools.partial(
        pl.kernel,
        mesh=_mesh,
        compiler_params=pltpu.CompilerParams(use_tc_tiling_on_sc=True),
        out_type=jax.ShapeDtypeStruct((B * F,), jnp.float32),
        scratch_types=[
            pltpu.VMEM((IPW,), jnp.int32),
            pltpu.VMEM((L, 2 * F), jnp.float32),
            pltpu.VMEM((L, 2 * F), jnp.float32),
            pltpu.VMEM((BPW * F,), jnp.float32),
            pltpu.SMEM((L,), jnp.int32),
            pltpu.SMEM((L,), jnp.int32),
            pltpu.SemaphoreType.DMA,
            pltpu.SemaphoreType.DMA,
            pltpu.SemaphoreType.DMA,
        ],
    )(fn)


_gather_lo = _make_gather(0, LOBLK, LOBLK, with_partial=False)
_gather_hi = _make_gather(LOBLK, NBLK, NBLK - LOBLK, with_partial=True)


def _tail_tc(ps_ref, w_ref, b_ref, bg_ref, bb_ref, lg_ref, lb_ref, o_ref):
    pooled = ps_ref[...] * (1.0 / L)
    h = lax.dot_general(pooled, w_ref[...], (((1,), (1,)), ((), ())),
                        preferred_element_type=jnp.float32) + b_ref[...]
    mu = jnp.mean(h, axis=0, keepdims=True)
    var = jnp.mean((h - mu) ** 2, axis=0, keepdims=True)
    h = (h - mu) * lax.rsqrt(var + EPS) * bg_ref[...] + bb_ref[...]
    lmu = jnp.mean(h, axis=-1, keepdims=True)
    lvar = jnp.mean((h - lmu) ** 2, axis=-1, keepdims=True)
    o_ref[...] = (h - lmu) * lax.rsqrt(lvar + EPS) * lg_ref[...] + lb_ref[...]


def kernel(x, table, W, b, bn_gamma, bn_beta, ln_gamma, ln_beta):
    xf = x.reshape(B * L).astype(jnp.int32)
    tableT = table.T
    t_lo = _pack(tableT, LOBLK, 0)
    t_hi = _pack(tableT, NBLK - LOBLK, LOBLK)
    ps_lo = _gather_lo(xf, t_lo)
    ps_hi = _gather_hi(xf, t_hi, ps_lo)
    out = pl.pallas_call(
        _tail_tc,
        out_shape=jax.ShapeDtypeStruct((B, F), jnp.float32),
    )(ps_hi.reshape(B, F), W, b.reshape(1, F), bn_gamma.reshape(1, F),
      bn_beta.reshape(1, F), ln_gamma.reshape(1, F), ln_beta.reshape(1, F))
    return out


# speedup vs baseline: 12.4806x; 12.4806x over previous
import functools

import jax
import jax.numpy as jnp
from jax import lax
from jax.experimental import pallas as pl
from jax.experimental.pallas import tpu as pltpu
from jax.experimental.pallas import tpu_sc as plsc

B = 4096
L = 200
F = 64
VOCAB = 1000000
EPS = 1e-5

NC = 2
NS = 16
NW = NC * NS
BPW = B // NW
IPW = BPW * L

VB = 32768
HVB = VB // 2
NBLK = (VOCAB + VB - 1) // VB
LOBLK = 16
ZPAD = 32

_mesh = plsc.VectorSubcoreMesh(core_axis_name="c", subcore_axis_name="s")


def _pack_body(lastpid):
    def f(tt_ref, o_ref):
        t = jnp.transpose(tt_ref[...])
        th = jnp.concatenate([t[:HVB, :], t[HVB:, :]], axis=1)
        o_ref[...] = jnp.where(pl.program_id(0) == lastpid, 0.0, th)
    return f


def _pack(tableT, nblk, blk0):
    return pl.pallas_call(
        _pack_body(nblk),
        grid=(nblk + 1,),
        in_specs=[pl.BlockSpec(
            (F, VB), lambda j: (0, jnp.minimum(blk0 + j, NBLK - 1)))],
        out_specs=pl.BlockSpec((HVB, 2 * F), lambda j: (j, 0)),
        out_shape=jax.ShapeDtypeStruct((nblk * HVB + ZPAD, 2 * F),
                                       jnp.float32),
    )(tableT)


def _make_gather(loblk, hiblk, nblk, with_partial):
    zrow = nblk * HVB

    def body(refs):
        if with_partial:
            (x_hbm, table_hbm, pin_hbm, out_hbm, idx_v, buf0, buf1, out_v,
             par0, par1, semi, sem0, sem1) = refs
        else:
            (x_hbm, table_hbm, out_hbm, idx_v, buf0, buf1, out_v,
             par0, par1, semi, sem0, sem1) = refs
        wid = lax.axis_index("s") * NC + lax.axis_index("c")
        zr = zrow + wid
        pltpu.async_copy(x_hbm.at[pl.ds(wid * IPW, IPW)], idx_v, semi).wait()
        if with_partial:
            pltpu.async_copy(pin_hbm.at[pl.ds(wid * BPW * F, BPW * F)],
                             out_v, semi).wait()

        zero = jnp.zeros((16,), jnp.float32)

        def enqueue(r, buf, par, sem):
            base = r * L

            def one(idx, slot):
                h = lax.shift_right_logical(idx, 15)
                valid = jnp.logical_and(h >= loblk, h < hiblk)
                q = (h - loblk) * HVB + (idx & (HVB - 1))
                qe = jnp.where(valid, q, zr)
                pltpu.async_copy(table_hbm.at[pl.ds(qe, 1)],
                                 buf.at[pl.ds(slot, 1)], sem)
                par[slot] = (lax.shift_right_logical(idx, 14) & 1) * F

            def ek(k, _):
                vec = idx_v[pl.ds(base + k * 16, 16)]
                for t in range(16):
                    one(vec[t], k * 16 + t)
                return 0
            lax.fori_loop(0, (L // 16), ek, 0)
            vec = idx_v[pl.ds(base + L - 16, 16)]
            for t in range(16 - (L % 16), 16):
                one(vec[t], L - 16 + t)

        def drain(buf, sem):
            pltpu.make_async_copy(table_hbm.at[pl.ds(0, L)], buf, sem).wait()

        def accumulate(buf, par, r):
            def jbody(j, carry):
                a0, a1, a2, a3 = carry
                row = buf.at[j]
                off = par[j]
                return (a0 + row[pl.ds(off, 16)],
                        a1 + row[pl.ds(off + 16, 16)],
                        a2 + row[pl.ds(off + 32, 16)],
                        a3 + row[pl.ds(off + 48, 16)])
            if with_partial:
                init = (out_v[pl.ds(r * F, 16)],
                        out_v[pl.ds(r * F + 16, 16)],
                        out_v[pl.ds(r * F + 32, 16)],
                        out_v[pl.ds(r * F + 48, 16)])
            else:
                init = (zero, zero, zero, zero)
            a0, a1, a2, a3 = lax.fori_loop(0, L, jbody, init)
            out_v[pl.ds(r * F, 16)] = a0
            out_v[pl.ds(r * F + 16, 16)] = a1
            out_v[pl.ds(r * F + 32, 16)] = a2
            out_v[pl.ds(r * F + 48, 16)] = a3

        enqueue(0, buf0, par0, sem0)

        def loop(i, _):
            r0 = 2 * i
            enqueue(r0 + 1, buf1, par1, sem1)
            drain(buf0, sem0)
            accumulate(buf0, par0, r0)

            @pl.when(r0 + 2 < BPW)
            def _():
                enqueue(r0 + 2, buf0, par0, sem0)
            drain(buf1, sem1)
            accumulate(buf1, par1, r0 + 1)
            return 0

        lax.fori_loop(0, BPW // 2, loop, 0)
        pltpu.sync_copy(out_v, out_hbm.at[pl.ds(wid * BPW * F, BPW * F)])

    def fn(*refs):
        body(refs)

    return functools.partial(
        pl.kernel,
        mesh=_mesh,
        compiler_params=pltpu.CompilerParams(use_tc_tiling_on_sc=True),
        out_type=jax.ShapeDtypeStruct((B * F,), jnp.float32),
        scratch_types=[
            pltpu.VMEM((IPW,), jnp.int32),
            pltpu.VMEM((L, 2 * F), jnp.float32),
            pltpu.VMEM((L, 2 * F), jnp.float32),
            pltpu.VMEM((BPW * F,), jnp.float32),
            pltpu.SMEM((L,), jnp.int32),
            pltpu.SMEM((L,), jnp.int32),
            pltpu.SemaphoreType.DMA,
            pltpu.SemaphoreType.DMA,
            pltpu.SemaphoreType.DMA,
        ],
    )(fn)


_gather_lo = _make_gather(0, LOBLK, LOBLK, with_partial=False)
_gather_hi = _make_gather(LOBLK, NBLK, NBLK - LOBLK, with_partial=True)


def _tail_tc(ps_ref, w_ref, b_ref, bg_ref, bb_ref, lg_ref, lb_ref, o_ref):
    pooled = ps_ref[...] * (1.0 / L)
    h = lax.dot_general(pooled, w_ref[...], (((1,), (1,)), ((), ())),
                        preferred_element_type=jnp.float32) + b_ref[...]
    mu = jnp.mean(h, axis=0, keepdims=True)
    var = jnp.mean((h - mu) ** 2, axis=0, keepdims=True)
    h = (h - mu) * lax.rsqrt(var + EPS) * bg_ref[...] + bb_ref[...]
    lmu = jnp.mean(h, axis=-1, keepdims=True)
    lvar = jnp.mean((h - lmu) ** 2, axis=-1, keepdims=True)
    o_ref[...] = (h - lmu) * lax.rsqrt(lvar + EPS) * lg_ref[...] + lb_ref[...]


def kernel(x, table, W, b, bn_gamma, bn_beta, ln_gamma, ln_beta):
    xf = x.reshape(B * L).astype(jnp.int32)
    tableT = table.T
    t_lo = _pack(tableT, LOBLK, 0)
    t_hi = _pack(tableT, NBLK - LOBLK, LOBLK)
    ps_lo = _gather_lo(xf, t_lo)
    ps_hi = _gather_hi(xf, t_hi, ps_lo)
    out = pl.pallas_call(
        _tail_tc,
        out_shape=jax.ShapeDtypeStruct((B, F), jnp.float32),
    )(ps_hi.reshape(B, F), W, b.reshape(1, F), bn_gamma.reshape(1, F),
      bn_beta.reshape(1, F), ln_gamma.reshape(1, F), ln_beta.reshape(1, F))
    return out


# speedup vs baseline: 35.1279x; 2.8146x over previous
import functools

import jax
import jax.numpy as jnp
from jax import lax
from jax.experimental import pallas as pl
from jax.experimental.pallas import tpu as pltpu
from jax.experimental.pallas import tpu_sc as plsc

B = 4096
L = 200
F = 64
VOCAB = 1000000
EPS = 1e-5

NC = 2
NS = 16
NW = NC * NS
BPW = B // NW
IPW = BPW * L

VB = 32768
HVB = VB // 2
NBLK = (VOCAB + VB - 1) // VB
LOBLK = 16
ZPAD = 32

_mesh = plsc.VectorSubcoreMesh(core_axis_name="c", subcore_axis_name="s")


def _pack_body(lastpid):
    def f(tt_ref, o_ref):
        t = jnp.transpose(tt_ref[...])
        th = jnp.concatenate([t[:HVB, :], t[HVB:, :]], axis=1)
        o_ref[...] = jnp.where(pl.program_id(0) == lastpid, 0.0, th)
    return f


def _pack(tableT, nblk, blk0):
    return pl.pallas_call(
        _pack_body(nblk),
        grid=(nblk + 1,),
        in_specs=[pl.BlockSpec(
            (F, VB), lambda j: (0, jnp.minimum(blk0 + j, NBLK - 1)))],
        out_specs=pl.BlockSpec((HVB, 2 * F), lambda j: (j, 0)),
        out_shape=jax.ShapeDtypeStruct((nblk * HVB + ZPAD, 2 * F),
                                       jnp.float32),
    )(tableT)


def _make_gather(loblk, hiblk, nblk, with_partial):
    zrow = nblk * HVB

    def body(refs):
        if with_partial:
            (x_hbm, table_hbm, pin_hbm, out_hbm, idx_v, buf0, buf1, out_v,
             par0, par1, nvw, semi, sem0, sem1) = refs
        else:
            (x_hbm, table_hbm, out_hbm, idx_v, buf0, buf1, out_v,
             par0, par1, nvw, semi, sem0, sem1) = refs
        wid = lax.axis_index("s") * NC + lax.axis_index("c")
        zr = zrow + wid
        pltpu.async_copy(x_hbm.at[pl.ds(wid * IPW, IPW)], idx_v, semi).wait()
        if with_partial:
            pltpu.async_copy(pin_hbm.at[pl.ds(wid * BPW * F, BPW * F)],
                             out_v, semi).wait()

        zero = jnp.zeros((16,), jnp.float32)

        def enqueue(r, buf, par, sem, nslot):
            base = r * L

            def one(idx, slot, nv):
                h = lax.shift_right_logical(idx, 15)
                valid = jnp.logical_and(h >= loblk, h < hiblk)
                q = (h - loblk) * HVB + (idx & (HVB - 1))
                off = (lax.shift_right_logical(idx, 14) & 1) * F
                par[slot] = off

                @pl.when(valid)
                def _():
                    pltpu.async_copy(table_hbm.at[pl.ds(q, 1)],
                                     buf.at[pl.ds(slot, 1)], sem)

                @pl.when(jnp.logical_not(valid))
                def _():
                    row = buf.at[slot]
                    row[pl.ds(off, 16)] = zero
                    row[pl.ds(off + 16, 16)] = zero
                    row[pl.ds(off + 32, 16)] = zero
                    row[pl.ds(off + 48, 16)] = zero
                return nv + jnp.where(valid, 1, 0)

            def ek(k, nv):
                vec = idx_v[pl.ds(base + k * 16, 16)]
                for t in range(16):
                    nv = one(vec[t], k * 16 + t, nv)
                return nv
            nv = lax.fori_loop(0, (L // 16), ek, jnp.int32(0))
            vec = idx_v[pl.ds(base + L - 16, 16)]
            for t in range(16 - (L % 16), 16):
                nv = one(vec[t], L - 16 + t, nv)
            nvw[nslot] = nv

        def drain(buf, sem, nslot):
            def w(i, _):
                pltpu.make_async_copy(table_hbm.at[pl.ds(0, 1)],
                                      buf.at[pl.ds(0, 1)], sem).wait()
                return 0
            lax.fori_loop(0, nvw[nslot], w, 0)

        def accumulate(buf, par, r):
            def jbody(j, carry):
                a0, a1, a2, a3 = carry
                row = buf.at[j]
                off = par[j]
                return (a0 + row[pl.ds(off, 16)],
                        a1 + row[pl.ds(off + 16, 16)],
                        a2 + row[pl.ds(off + 32, 16)],
                        a3 + row[pl.ds(off + 48, 16)])
            if with_partial:
                init = (out_v[pl.ds(r * F, 16)],
                        out_v[pl.ds(r * F + 16, 16)],
                        out_v[pl.ds(r * F + 32, 16)],
                        out_v[pl.ds(r * F + 48, 16)])
            else:
                init = (zero, zero, zero, zero)
            a0, a1, a2, a3 = lax.fori_loop(0, L, jbody, init)
            out_v[pl.ds(r * F, 16)] = a0
            out_v[pl.ds(r * F + 16, 16)] = a1
            out_v[pl.ds(r * F + 32, 16)] = a2
            out_v[pl.ds(r * F + 48, 16)] = a3

        enqueue(0, buf0, par0, sem0, 0)

        def loop(i, _):
            r0 = 2 * i
            enqueue(r0 + 1, buf1, par1, sem1, 1)
            drain(buf0, sem0, 0)
            accumulate(buf0, par0, r0)
            enqueue(r0 + 2, buf0, par0, sem0, 0)
            drain(buf1, sem1, 1)
            accumulate(buf1, par1, r0 + 1)
            return 0

        lax.fori_loop(0, BPW // 2 - 1, loop, 0)
        enqueue(BPW - 1, buf1, par1, sem1, 1)
        drain(buf0, sem0, 0)
        accumulate(buf0, par0, BPW - 2)
        drain(buf1, sem1, 1)
        accumulate(buf1, par1, BPW - 1)
        pltpu.sync_copy(out_v, out_hbm.at[pl.ds(wid * BPW * F, BPW * F)])

    def fn(*refs):
        body(refs)

    return functools.partial(
        pl.kernel,
        mesh=_mesh,
        compiler_params=pltpu.CompilerParams(use_tc_tiling_on_sc=True),
        out_type=jax.ShapeDtypeStruct((B * F,), jnp.float32),
        scratch_types=[
            pltpu.VMEM((IPW,), jnp.int32),
            pltpu.VMEM((L, 2 * F), jnp.float32),
            pltpu.VMEM((L, 2 * F), jnp.float32),
            pltpu.VMEM((BPW * F,), jnp.float32),
            pltpu.SMEM((L,), jnp.int32),
            pltpu.SMEM((L,), jnp.int32),
            pltpu.SMEM((8,), jnp.int32),
            pltpu.SemaphoreType.DMA,
            pltpu.SemaphoreType.DMA,
            pltpu.SemaphoreType.DMA,
        ],
    )(fn)


_gather_lo = _make_gather(0, LOBLK, LOBLK, with_partial=False)
_gather_hi = _make_gather(LOBLK, NBLK, NBLK - LOBLK, with_partial=True)


def _tail_tc(ps_ref, w_ref, b_ref, bg_ref, bb_ref, lg_ref, lb_ref, o_ref):
    pooled = ps_ref[...] * (1.0 / L)
    h = lax.dot_general(pooled, w_ref[...], (((1,), (1,)), ((), ())),
                        preferred_element_type=jnp.float32) + b_ref[...]
    mu = jnp.mean(h, axis=0, keepdims=True)
    var = jnp.mean((h - mu) ** 2, axis=0, keepdims=True)
    h = (h - mu) * lax.rsqrt(var + EPS) * bg_ref[...] + bb_ref[...]
    lmu = jnp.mean(h, axis=-1, keepdims=True)
    lvar = jnp.mean((h - lmu) ** 2, axis=-1, keepdims=True)
    o_ref[...] = (h - lmu) * lax.rsqrt(lvar + EPS) * lg_ref[...] + lb_ref[...]


def kernel(x, table, W, b, bn_gamma, bn_beta, ln_gamma, ln_beta):
    xf = x.reshape(B * L).astype(jnp.int32)
    tableT = table.T
    t_lo = _pack(tableT, LOBLK, 0)
    t_hi = _pack(tableT, NBLK - LOBLK, LOBLK)
    ps_lo = _gather_lo(xf, t_lo)
    ps_hi = _gather_hi(xf, t_hi, ps_lo)
    out = pl.pallas_call(
        _tail_tc,
        out_shape=jax.ShapeDtypeStruct((B, F), jnp.float32),
    )(ps_hi.reshape(B, F), W, b.reshape(1, F), bn_gamma.reshape(1, F),
      bn_beta.reshape(1, F), ln_gamma.reshape(1, F), ln_beta.reshape(1, F))
    return out
